# Initial kernel scaffold; baseline (speedup 1.0000x reference)
import jax
import jax.numpy as jnp
from jax.experimental import pallas as pl


def kernel(x, edge_index, adj_values, W, b):
    raise NotImplementedError("write your pallas kernel here")


# speedup vs baseline: 4.3696x; 4.3696x over previous
import dataclasses
import functools

import jax
import jax.numpy as jnp
from jax import lax
from jax.experimental import pallas as pl
from jax.experimental.pallas import tpu as pltpu
from jax.experimental.pallas import tpu_sc as plsc

N_NODES = 10000
N_EDGES = 320000
D = 128

NC = 2
NS = 16
L = 16
NW = NC * NS

CHUNK = 128
N_CHUNKS = N_EDGES // CHUNK
T_MAX = (N_CHUNKS + NW - 1) // NW
RBLK = 80
N_RBLK = N_NODES // RBLK
RB_T = (N_RBLK + NS - 1) // NS


def _matmul_body(x_ref, w_ref, o_ref):
    o_ref[...] = jnp.dot(x_ref[...], w_ref[...],
                         preferred_element_type=jnp.float32)


def _combine_body(p_ref, b_ref, o_ref):
    o_ref[...] = p_ref[0] + p_ref[1] + b_ref[...]


def _sc_spmm(h, row, col, val):
    mesh = plsc.VectorSubcoreMesh(core_axis_name="c", subcore_axis_name="s")
    cp = pltpu.CompilerParams()
    if "needs_layout_passes" in pltpu.CompilerParams.__dataclass_fields__:
        cp = dataclasses.replace(cp, needs_layout_passes=False)

    @functools.partial(
        pl.kernel,
        compiler_params=cp,
        out_type=jax.ShapeDtypeStruct((NC, N_NODES, D), jnp.float32),
        mesh=mesh,
        scratch_types=[
            pltpu.VMEM((CHUNK,), jnp.int32),
            pltpu.VMEM((CHUNK,), jnp.int32),
            pltpu.VMEM((CHUNK,), jnp.float32),
            pltpu.VMEM((CHUNK, D), jnp.float32),
            pltpu.VMEM_SHARED((N_NODES, D), jnp.float32),
            pltpu.SemaphoreType.DMA,
        ],
    )
    def spmm_kernel(h_hbm, row_hbm, col_hbm, val_hbm, out_hbm,
                    col_v, row_v, val_v, rows_v, acc_sh, sem):
        cid = lax.axis_index("c")
        sid = lax.axis_index("s")
        wid = sid * NC + cid

        @pl.loop(0, RBLK)
        def _(e):
            for k in range(D // L):
                rows_v[e, pl.ds(k * L, L)] = jnp.zeros((L,), jnp.float32)

        @pl.loop(0, RB_T)
        def _(t):
            blk = sid + t * NS

            @pl.when(blk < N_RBLK)
            def _():
                pltpu.sync_copy(rows_v.at[pl.ds(0, RBLK)],
                                acc_sh.at[pl.ds(blk * RBLK, RBLK)])

        plsc.subcore_barrier()

        @pl.loop(0, T_MAX)
        def _(t):
            m = wid + t * NW

            @pl.when(m < N_CHUNKS)
            def _():
                off = m * CHUNK
                pltpu.sync_copy(col_hbm.at[pl.ds(off, CHUNK)], col_v)
                pltpu.sync_copy(row_hbm.at[pl.ds(off, CHUNK)], row_v)
                pltpu.sync_copy(val_hbm.at[pl.ds(off, CHUNK)], val_v)
                pltpu.async_copy(h_hbm.at[col_v], rows_v, sem).wait()

                @pl.loop(0, CHUNK)
                def _(e):
                    bcast = plsc.load_gather(
                        val_v, [jnp.full((L,), e, jnp.int32)])
                    for k in range(D // L):
                        sl = pl.ds(k * L, L)
                        rows_v[e, sl] = rows_v[e, sl] * bcast

                pltpu.sync_copy(rows_v, acc_sh.at[row_v], add=True)

        plsc.subcore_barrier()

        @pl.loop(0, RB_T)
        def _(t):
            blk = sid + t * NS

            @pl.when(blk < N_RBLK)
            def _():
                pltpu.sync_copy(
                    acc_sh.at[pl.ds(blk * RBLK, RBLK)],
                    out_hbm.at[cid, pl.ds(blk * RBLK, RBLK)])

    return spmm_kernel(h, row, col, val)


def kernel(x, edge_index, adj_values, W, b):
    row = edge_index[0].astype(jnp.int32)
    col = edge_index[1].astype(jnp.int32)
    val = adj_values.astype(jnp.float32)

    h = pl.pallas_call(
        _matmul_body,
        grid=(10,),
        in_specs=[
            pl.BlockSpec((N_NODES // 10, D), lambda i: (i, 0)),
            pl.BlockSpec((D, D), lambda i: (0, 0)),
        ],
        out_specs=pl.BlockSpec((N_NODES // 10, D), lambda i: (i, 0)),
        out_shape=jax.ShapeDtypeStruct((N_NODES, D), jnp.float32),
    )(x, W)

    partials = _sc_spmm(h, row, col, val)

    b2 = b.reshape(1, D).astype(jnp.float32)
    out = pl.pallas_call(
        _combine_body,
        grid=(10,),
        in_specs=[
            pl.BlockSpec((NC, N_NODES // 10, D), lambda i: (0, i, 0)),
            pl.BlockSpec((1, D), lambda i: (0, 0)),
        ],
        out_specs=pl.BlockSpec((N_NODES // 10, D), lambda i: (i, 0)),
        out_shape=jax.ShapeDtypeStruct((N_NODES, D), jnp.float32),
    )(partials, b2)
    return out
